# Initial kernel scaffold; baseline (speedup 1.0000x reference)
import jax
import jax.numpy as jnp
from jax.experimental import pallas as pl


def kernel(x, edge_index, W1, b1, W2, b2):
    raise NotImplementedError("write your pallas kernel here")


# speedup vs baseline: 13.6695x; 13.6695x over previous
import dataclasses
import functools

import jax
import jax.numpy as jnp
from jax import lax
from jax.experimental import pallas as pl
from jax.experimental.pallas import tpu as pltpu
from jax.experimental.pallas import tpu_sc as plsc

N = 10000
DI = 128
DH = 256
DO = 128
NC = 2
NS = 16
NW = NC * NS
NPAD = 10240
RPT = NPAD // NS
K = 128

_MESH = plsc.VectorSubcoreMesh(core_axis_name="c", subcore_axis_name="s")


def _sc_params():
    cp = pltpu.CompilerParams()
    if "needs_layout_passes" in pltpu.CompilerParams.__dataclass_fields__:
        cp = dataclasses.replace(cp, needs_layout_passes=False)
    return cp


def _deg_call(dst4, nb):
    @functools.partial(
        pl.kernel,
        out_type=jax.ShapeDtypeStruct((NC, NPAD), jnp.float32),
        mesh=_MESH,
        compiler_params=_sc_params(),
        scratch_types=[
            pltpu.VMEM((nb, K), jnp.int32),
            pltpu.VMEM((NPAD,), jnp.float32),
            pltpu.VMEM_SHARED((NS, NPAD), jnp.float32),
            pltpu.VMEM((NS, RPT), jnp.float32),
            pltpu.VMEM((RPT,), jnp.float32),
            pltpu.SemaphoreType.DMA,
        ],
    )
    def deg_kernel(dst_hbm, deg_hbm, dst_v, deg_v, slots, chunk_v, res_v, sem):
        c = lax.axis_index("c")
        s = lax.axis_index("s")
        zeros16 = jnp.zeros((16,), jnp.float32)
        ones16 = jnp.ones((16,), jnp.float32)

        @pl.loop(0, NPAD, step=16)
        def _(i):
            deg_v[pl.ds(i, 16)] = zeros16

        pltpu.sync_copy(dst_hbm.at[c, s], dst_v)

        @pl.loop(0, nb)
        def _(r):
            @pl.loop(0, K, step=16)
            def _(k):
                idx = dst_v[r, pl.ds(k, 16)]
                plsc.addupdate_scatter(deg_v, [idx], ones16)

        pltpu.sync_copy(deg_v, slots.at[s])
        plsc.subcore_barrier()

        r0 = s * RPT
        pltpu.sync_copy(slots.at[:, pl.ds(r0, RPT)], chunk_v)

        @pl.loop(0, RPT, step=16)
        def _(i):
            acc = chunk_v[0, pl.ds(i, 16)]
            for t in range(1, NS):
                acc = acc + chunk_v[t, pl.ds(i, 16)]
            res_v[pl.ds(i, 16)] = acc

        pltpu.sync_copy(res_v, deg_hbm.at[c, pl.ds(r0, RPT)])

    return deg_kernel(dst4)


def _spmm_call(src4, dst4, y, nb):
    @functools.partial(
        pl.kernel,
        out_type=jax.ShapeDtypeStruct((NC, NPAD, DI), jnp.float32),
        mesh=_MESH,
        compiler_params=_sc_params(),
        scratch_types=[
            pltpu.VMEM((K,), jnp.int32),
            pltpu.VMEM((K,), jnp.int32),
            pltpu.VMEM((K, DI), jnp.float32),
            pltpu.VMEM_SHARED((NPAD, DI), jnp.float32),
            pltpu.SemaphoreType.DMA,
        ],
    )
    def spmm_kernel(src_hbm, dst_hbm, y_hbm, p_hbm, src_v, dst_v, rows_v, acc, sem):
        c = lax.axis_index("c")
        s = lax.axis_index("s")
        zeros16 = jnp.zeros((16,), jnp.float32)

        @pl.loop(0, K)
        def _(r):
            @pl.loop(0, DI, step=16)
            def _(k):
                rows_v[r, pl.ds(k, 16)] = zeros16

        @pl.loop(0, RPT, step=K)
        def _(m):
            pltpu.sync_copy(rows_v, acc.at[pl.ds(s * RPT + m, K)])

        plsc.subcore_barrier()

        @pl.loop(0, nb)
        def _(j):
            pltpu.sync_copy(src_hbm.at[c, s, j], src_v)
            pltpu.sync_copy(dst_hbm.at[c, s, j], dst_v)
            pltpu.async_copy(y_hbm.at[src_v], rows_v, sem).wait()
            pltpu.sync_copy(rows_v, acc.at[dst_v], add=True)

        plsc.subcore_barrier()

        @pl.loop(0, RPT, step=K)
        def _(m):
            pltpu.sync_copy(acc.at[pl.ds(s * RPT + m, K)], rows_v)
            pltpu.sync_copy(rows_v, p_hbm.at[c, pl.ds(s * RPT + m, K)])

    return spmm_kernel(src4, dst4, y)


def _tc1_call(x_pad, degs_t):
    br = 2048
    grid = (NPAD // br,)

    def body(x_ref, d_ref, y1_ref, dinv_ref):
        deg = d_ref[:, 0:1] + d_ref[:, 1:2] + 1.0
        dinv = lax.rsqrt(deg)
        dinv_ref[...] = dinv
        y1_ref[...] = x_ref[...] * dinv

    return pl.pallas_call(
        body,
        grid=grid,
        in_specs=[
            pl.BlockSpec((br, DI), lambda i: (i, 0)),
            pl.BlockSpec((br, NC), lambda i: (i, 0)),
        ],
        out_specs=[
            pl.BlockSpec((br, DI), lambda i: (i, 0)),
            pl.BlockSpec((br, 1), lambda i: (i, 0)),
        ],
        out_shape=[
            jax.ShapeDtypeStruct((NPAD, DI), jnp.float32),
            jax.ShapeDtypeStruct((NPAD, 1), jnp.float32),
        ],
    )(x_pad, degs_t)


def _tc2_call(p, y1, dinv, W1, b1, W2, b2):
    br = 1024
    grid = (NPAD // br,)

    def body(p_ref, y1_ref, dinv_ref, w1_ref, b1_ref, w2_ref, y2_ref):
        dv = dinv_ref[...]
        a = (p_ref[0] + p_ref[1] + y1_ref[...]) * dv
        h = jnp.dot(a, w1_ref[...], preferred_element_type=jnp.float32)
        h = jnp.maximum(h + b1_ref[...], 0.0)
        t = jnp.dot(h, w2_ref[...], preferred_element_type=jnp.float32)
        y2_ref[...] = t * dv

    return pl.pallas_call(
        body,
        grid=grid,
        in_specs=[
            pl.BlockSpec((NC, br, DI), lambda i: (0, i, 0)),
            pl.BlockSpec((br, DI), lambda i: (i, 0)),
            pl.BlockSpec((br, 1), lambda i: (i, 0)),
            pl.BlockSpec((DI, DH), lambda i: (0, 0)),
            pl.BlockSpec((1, DH), lambda i: (0, 0)),
            pl.BlockSpec((DH, DO), lambda i: (0, 0)),
        ],
        out_specs=pl.BlockSpec((br, DO), lambda i: (i, 0)),
        out_shape=jax.ShapeDtypeStruct((NPAD, DO), jnp.float32),
    )(p, y1, dinv, W1, b1, W2)


def _tc3_call(p2, y2, dinv, b2):
    br = 2048
    grid = (NPAD // br,)

    def body(p_ref, y2_ref, dinv_ref, b2_ref, z_ref):
        dv = dinv_ref[...]
        z = (p_ref[0] + p_ref[1] + y2_ref[...]) * dv + b2_ref[...]
        z_ref[...] = jnp.maximum(z, 0.0)

    return pl.pallas_call(
        body,
        grid=grid,
        in_specs=[
            pl.BlockSpec((NC, br, DO), lambda i: (0, i, 0)),
            pl.BlockSpec((br, DO), lambda i: (i, 0)),
            pl.BlockSpec((br, 1), lambda i: (i, 0)),
            pl.BlockSpec((1, DO), lambda i: (0, 0)),
        ],
        out_specs=pl.BlockSpec((br, DO), lambda i: (i, 0)),
        out_shape=jax.ShapeDtypeStruct((NPAD, DO), jnp.float32),
    )(p2, y2, dinv, b2)


def kernel(x, edge_index, W1, b1, W2, b2):
    e = edge_index.shape[1]
    assert e % NW == 0
    ew = e // NW
    nb = (ew + K - 1) // K
    ewp = nb * K

    src = edge_index[0].astype(jnp.int32).reshape(NW, ew)
    dst = edge_index[1].astype(jnp.int32).reshape(NW, ew)
    if ewp != ew:
        pad = jnp.full((NW, ewp - ew), NPAD - 1, jnp.int32)
        src = jnp.concatenate([src, pad], axis=1)
        dst = jnp.concatenate([dst, pad], axis=1)
    src4 = src.reshape(NC, NS, nb, K)
    dst4 = dst.reshape(NC, NS, nb, K)

    x_pad = jnp.pad(x, ((0, NPAD - N), (0, 0)))
    b1r = b1.reshape(1, DH)
    b2r = b2.reshape(1, DO)

    degs = _deg_call(dst4, nb)
    y1, dinv = _tc1_call(x_pad, degs.T)
    p1 = _spmm_call(src4, dst4, y1, nb)
    y2 = _tc2_call(p1, y1, dinv, W1, b1r, W2, b2r)
    p2 = _spmm_call(src4, dst4, y2, nb)
    z = _tc3_call(p2, y2, dinv, b2r)
    return z[:N]
